# Initial kernel scaffold; baseline (speedup 1.0000x reference)
import jax
import jax.numpy as jnp
from jax.experimental import pallas as pl


def kernel(dense, sparse_0, sparse_1, sparse_2, sparse_3, sparse_4, sparse_5, sparse_6, sparse_7, sparse_8, sparse_9, sparse_10, sparse_11, sparse_12, sparse_13, sparse_14, sparse_15, sparse_16, sparse_17, sparse_18, sparse_19, sparse_20, sparse_21, sparse_22, sparse_23, sparse_24, sparse_25, table_0, table_1, table_2, table_3, table_4, table_5, table_6, table_7, table_8, table_9, table_10, table_11, table_12, table_13, table_14, table_15, table_16, table_17, table_18, table_19, table_20, table_21, table_22, table_23, table_24, table_25, bw_0, bb_0, bw_1, bb_1, bw_2, bb_2, u_0, v_0, dcnb_0, u_1, v_1, dcnb_1, u_2, v_2, dcnb_2, tw_0, tb_0, tw_1, tb_1, tw_2, tb_2, tw_3, tb_3, tw_4, tb_4):
    raise NotImplementedError("write your pallas kernel here")


# speedup vs baseline: 4.6975x; 4.6975x over previous
import functools

import jax
import jax.numpy as jnp
from jax import lax
from jax.experimental import pallas as pl
from jax.experimental.pallas import tpu as pltpu
from jax.experimental.pallas import tpu_sc as plsc

_B = 4096
_EMB = 128
_NS = 26
_MH = [3, 2, 1, 2, 6, 1, 1, 1, 1, 7, 3, 8, 1, 6, 9, 5, 1, 1, 1, 12, 100,
       27, 10, 3, 1, 1]
_NW = 32
_BPW = _B // _NW
_D = 3456
_PREC = jax.lax.Precision.DEFAULT
_INTERP = False
_TC_PARAMS = pltpu.CompilerParams(vmem_limit_bytes=100 * 1024 * 1024)


def _chunk_cfg(mh):
    c = 1
    while c * 2 <= _BPW and (c * 2) * mh <= 128:
        c *= 2
    return c, _BPW // c


_CFG = [_chunk_cfg(m) for m in _MH]


def _sc_embed(idx_list, tables):
    mesh = plsc.VectorSubcoreMesh(core_axis_name="c", subcore_axis_name="s",
                                  num_cores=2, num_subcores=16)
    scratch = [pltpu.VMEM((nc, c * mh), jnp.int32)
               for (c, nc), mh in zip(_CFG, _MH)]
    scratch += [
        pltpu.VMEM((128, _EMB), jnp.float32),
        pltpu.VMEM((_BPW, _EMB), jnp.float32),
        pltpu.SemaphoreType.DMA,
    ]

    @functools.partial(
        pl.kernel,
        out_type=jax.ShapeDtypeStruct((_NS, _B, _EMB), jnp.float32),
        mesh=mesh,
        scratch_types=scratch,
        interpret=_INTERP,
    )
    def k(*refs):
        idx_hbm = refs[:_NS]
        tab_hbm = refs[_NS:2 * _NS]
        out_hbm = refs[2 * _NS]
        idx_v = refs[2 * _NS + 1: 2 * _NS + 1 + _NS]
        rows_v = refs[2 * _NS + 1 + _NS]
        out_v = refs[2 * _NS + 2 + _NS]
        sem = refs[2 * _NS + 3 + _NS]
        wid = lax.axis_index("s") * 2 + lax.axis_index("c")
        base = wid * _BPW
        for i in range(_NS):
            mh = _MH[i]
            c, nc = _CFG[i]
            rpc = c * mh
            pltpu.sync_copy(idx_hbm[i].at[wid], idx_v[i])
            if mh == 1:
                pltpu.async_copy(tab_hbm[i].at[idx_v[i].at[0]], out_v,
                                 sem).wait()
            else:
                inv = 1.0 / mh

                def chunk_body(kk, _, i=i, mh=mh, c=c, rpc=rpc, inv=inv):
                    pltpu.async_copy(tab_hbm[i].at[idx_v[i].at[kk]],
                                     rows_v.at[pl.ds(0, rpc)], sem).wait()

                    def samp_body(s, _2):
                        rb = s * mh

                        def mh_body(j, acc):
                            return tuple(
                                acc[g] + rows_v[rb + j, pl.ds(g * 16, 16)]
                                for g in range(8))

                        acc = lax.fori_loop(
                            0, mh, mh_body,
                            tuple(jnp.zeros((16,), jnp.float32)
                                  for _ in range(8)))
                        row = kk * c + s
                        for g in range(8):
                            out_v[row, pl.ds(g * 16, 16)] = acc[g] * inv
                        return 0

                    lax.fori_loop(0, c, samp_body, 0)
                    return 0

                lax.fori_loop(0, nc, chunk_body, 0)
            pltpu.sync_copy(out_v, out_hbm.at[i, pl.ds(base, _BPW)])

    return k(*idx_list, *tables)


def _full_spec(shape):
    nd = len(shape)
    return pl.BlockSpec(shape, lambda i, _nd=nd: (0,) * _nd)


def _dot(a, b):
    return jnp.dot(a, b, preferred_element_type=jnp.float32,
                   precision=_PREC)


def _bottom_mlp(dense, bw0, bb0, bw1, bb1, bw2, bb2):
    bm = 512

    def body(x_ref, w0, b0, w1, b1, w2, b2, o_ref):
        h = jnp.maximum(_dot(x_ref[...], w0[...]) + b0[...], 0.0)
        h = jnp.maximum(_dot(h, w1[...]) + b1[...], 0.0)
        h = jnp.maximum(_dot(h, w2[...]) + b2[...], 0.0)
        o_ref[...] = h

    consts = (bw0, bb0.reshape(1, -1), bw1, bb1.reshape(1, -1),
              bw2, bb2.reshape(1, -1))
    return pl.pallas_call(
        body,
        grid=(_B // bm,),
        in_specs=[pl.BlockSpec((bm, 13), lambda i: (i, 0))]
        + [_full_spec(a.shape) for a in consts],
        out_specs=pl.BlockSpec((bm, _EMB), lambda i: (i, 0)),
        out_shape=jax.ShapeDtypeStruct((_B, _EMB), jnp.float32),
        compiler_params=_TC_PARAMS,
        interpret=_INTERP,
    )(dense, *consts)


def _dcn(demb, emb26, u0, v0, c0, u1, v1, c1, u2, v2, c2):
    bm = 256

    def body(demb_ref, emb_ref, u0r, v0r, c0r, u1r, v1r, c1r, u2r, v2r,
             c2r, o_ref):
        x0 = jnp.concatenate(
            [demb_ref[...]] + [emb_ref[t] for t in range(_NS)], axis=1)
        xl = x0
        for u_r, v_r, c_r in ((u0r, v0r, c0r), (u1r, v1r, c1r),
                              (u2r, v2r, c2r)):
            uu = _dot(xl, u_r[...])
            vv = _dot(uu, v_r[...]) + c_r[...]
            xl = x0 * vv + xl
        o_ref[...] = xl

    consts = (u0, v0, c0.reshape(1, -1), u1, v1, c1.reshape(1, -1),
              u2, v2, c2.reshape(1, -1))
    return pl.pallas_call(
        body,
        grid=(_B // bm,),
        in_specs=[
            pl.BlockSpec((bm, _EMB), lambda i: (i, 0)),
            pl.BlockSpec((_NS, bm, _EMB), lambda i: (0, i, 0)),
        ] + [_full_spec(a.shape) for a in consts],
        out_specs=pl.BlockSpec((bm, _D), lambda i: (i, 0)),
        out_shape=jax.ShapeDtypeStruct((_B, _D), jnp.float32),
        compiler_params=_TC_PARAMS,
        interpret=_INTERP,
    )(demb, emb26, *consts)


def _top_mlp(xl, tw0, tb0, tw1, tb1, tw2, tb2, tw3, tb3, tw4, tb4):
    bm = 512
    tw4p = jnp.pad(tw4, ((0, 0), (0, 127)))
    tb4p = jnp.pad(tb4.reshape(1, -1), ((0, 0), (0, 127)))

    def body(x_ref, w0, b0, w1, b1, w2, b2, w3, b3, w4, b4, o_ref):
        h = jnp.maximum(_dot(x_ref[...], w0[...]) + b0[...], 0.0)
        h = jnp.maximum(_dot(h, w1[...]) + b1[...], 0.0)
        h = jnp.maximum(_dot(h, w2[...]) + b2[...], 0.0)
        h = jnp.maximum(_dot(h, w3[...]) + b3[...], 0.0)
        o_ref[...] = _dot(h, w4[...]) + b4[...]

    consts = (tw0, tb0.reshape(1, -1), tw1, tb1.reshape(1, -1),
              tw2, tb2.reshape(1, -1), tw3, tb3.reshape(1, -1),
              tw4p, tb4p)
    return pl.pallas_call(
        body,
        grid=(_B // bm,),
        in_specs=[pl.BlockSpec((bm, _D), lambda i: (i, 0))]
        + [_full_spec(a.shape) for a in consts],
        out_specs=pl.BlockSpec((bm, _EMB), lambda i: (i, 0)),
        out_shape=jax.ShapeDtypeStruct((_B, _EMB), jnp.float32),
        compiler_params=_TC_PARAMS,
        interpret=_INTERP,
    )(xl, *consts)


def kernel(dense, sparse_0, sparse_1, sparse_2, sparse_3, sparse_4,
           sparse_5, sparse_6, sparse_7, sparse_8, sparse_9, sparse_10,
           sparse_11, sparse_12, sparse_13, sparse_14, sparse_15,
           sparse_16, sparse_17, sparse_18, sparse_19, sparse_20,
           sparse_21, sparse_22, sparse_23, sparse_24, sparse_25,
           table_0, table_1, table_2, table_3, table_4, table_5, table_6,
           table_7, table_8, table_9, table_10, table_11, table_12,
           table_13, table_14, table_15, table_16, table_17, table_18,
           table_19, table_20, table_21, table_22, table_23, table_24,
           table_25,
           bw_0, bb_0, bw_1, bb_1, bw_2, bb_2,
           u_0, v_0, dcnb_0, u_1, v_1, dcnb_1, u_2, v_2, dcnb_2,
           tw_0, tb_0, tw_1, tb_1, tw_2, tb_2, tw_3, tb_3, tw_4, tb_4):
    sparses = [sparse_0, sparse_1, sparse_2, sparse_3, sparse_4, sparse_5,
               sparse_6, sparse_7, sparse_8, sparse_9, sparse_10,
               sparse_11, sparse_12, sparse_13, sparse_14, sparse_15,
               sparse_16, sparse_17, sparse_18, sparse_19, sparse_20,
               sparse_21, sparse_22, sparse_23, sparse_24, sparse_25]
    tables = [table_0, table_1, table_2, table_3, table_4, table_5,
              table_6, table_7, table_8, table_9, table_10, table_11,
              table_12, table_13, table_14, table_15, table_16, table_17,
              table_18, table_19, table_20, table_21, table_22, table_23,
              table_24, table_25]
    idx = [s.reshape(_NW, nc, c * mh)
           for s, (c, nc), mh in zip(sparses, _CFG, _MH)]
    emb26 = _sc_embed(idx, tables)
    demb = _bottom_mlp(dense, bw_0, bb_0, bw_1, bb_1, bw_2, bb_2)
    xl = _dcn(demb, emb26, u_0, v_0, dcnb_0, u_1, v_1, dcnb_1,
              u_2, v_2, dcnb_2)
    top = _top_mlp(xl, tw_0, tb_0, tw_1, tb_1, tw_2, tb_2, tw_3, tb_3,
                   tw_4, tb_4)
    return top[:, 0]


# speedup vs baseline: 7.4675x; 1.5897x over previous
import functools

import jax
import jax.numpy as jnp
from jax import lax
from jax.experimental import pallas as pl
from jax.experimental.pallas import tpu as pltpu
from jax.experimental.pallas import tpu_sc as plsc

_B = 4096
_EMB = 128
_NS = 26
_MH = [3, 2, 1, 2, 6, 1, 1, 1, 1, 7, 3, 8, 1, 6, 9, 5, 1, 1, 1, 12, 100,
       27, 10, 3, 1, 1]
_NW = 32
_BPW = _B // _NW
_D = 3456
_PREC = jax.lax.Precision.DEFAULT
_INTERP = False
_TC_PARAMS = pltpu.CompilerParams(vmem_limit_bytes=100 * 1024 * 1024)


def _chunk_cfg(mh):
    c = 1
    while c * 2 <= _BPW and (c * 2) * mh <= 128:
        c *= 2
    return c, _BPW // c


_CFG = [_chunk_cfg(m) for m in _MH]


_NBUF = 3


def _sc_embed(idx_list, tables):
    mesh = plsc.VectorSubcoreMesh(core_axis_name="c", subcore_axis_name="s",
                                  num_cores=2, num_subcores=16)
    scratch = [pltpu.VMEM((nc, c * mh), jnp.int32)
               for (c, nc), mh in zip(_CFG, _MH)]
    scratch += [
        pltpu.VMEM((_NBUF, 128, _EMB), jnp.float32),
        pltpu.VMEM((2, _BPW, _EMB), jnp.float32),
        pltpu.SemaphoreType.DMA((_NBUF,)),
        pltpu.SemaphoreType.DMA((2,)),
        pltpu.SemaphoreType.DMA((_NS,)),
    ]

    @functools.partial(
        pl.kernel,
        out_type=jax.ShapeDtypeStruct((_NS, _B, _EMB), jnp.float32),
        mesh=mesh,
        scratch_types=scratch,
        interpret=_INTERP,
    )
    def k(*refs):
        idx_hbm = refs[:_NS]
        tab_hbm = refs[_NS:2 * _NS]
        out_hbm = refs[2 * _NS]
        idx_v = refs[2 * _NS + 1: 2 * _NS + 1 + _NS]
        rows_v, out_v, gsem, osem, isem = refs[2 * _NS + 1 + _NS:]
        wid = lax.axis_index("s") * 2 + lax.axis_index("c")
        base = wid * _BPW
        for i in range(_NS):
            pltpu.async_copy(idx_hbm[i].at[wid], idx_v[i], isem.at[i])
        pending_out = [None, None]

        def drain_out(oslot):
            pi = pending_out[oslot]
            if pi is not None:
                pltpu.make_async_copy(
                    out_v.at[oslot],
                    out_hbm.at[pi, pl.ds(base, _BPW)],
                    osem.at[oslot]).wait()
                pending_out[oslot] = None

        for i in range(_NS):
            mh = _MH[i]
            c, nc = _CFG[i]
            rpc = c * mh
            oslot = i & 1
            pltpu.make_async_copy(idx_hbm[i].at[wid], idx_v[i],
                                  isem.at[i]).wait()
            drain_out(oslot)
            if mh == 1:
                pltpu.async_copy(tab_hbm[i].at[idx_v[i].at[0]],
                                 out_v.at[oslot], gsem.at[0]).wait()
            else:
                inv = 1.0 / mh
                for kk in range(min(_NBUF, nc)):
                    pltpu.async_copy(tab_hbm[i].at[idx_v[i].at[kk]],
                                     rows_v.at[kk, pl.ds(0, rpc)],
                                     gsem.at[kk])

                def chunk_body(kk, _, i=i, mh=mh, c=c, rpc=rpc, inv=inv,
                               nc=nc, oslot=oslot):
                    slot = lax.rem(kk, _NBUF)
                    pltpu.make_async_copy(
                        tab_hbm[i].at[idx_v[i].at[kk]],
                        rows_v.at[slot, pl.ds(0, rpc)],
                        gsem.at[slot]).wait()

                    def samp_body(s, _2):
                        rb = s * mh

                        def mh_body(j, acc):
                            return tuple(
                                acc[g] + rows_v[slot, rb + j,
                                                pl.ds(g * 16, 16)]
                                for g in range(8))

                        acc = lax.fori_loop(
                            0, mh, mh_body,
                            tuple(jnp.zeros((16,), jnp.float32)
                                  for _ in range(8)))
                        row = kk * c + s
                        for g in range(8):
                            out_v[oslot, row, pl.ds(g * 16, 16)] = (
                                acc[g] * inv)
                        return 0

                    lax.fori_loop(0, c, samp_body, 0)

                    @pl.when(kk + _NBUF < nc)
                    def _():
                        pltpu.async_copy(
                            tab_hbm[i].at[idx_v[i].at[kk + _NBUF]],
                            rows_v.at[slot, pl.ds(0, rpc)],
                            gsem.at[slot])

                    return 0

                lax.fori_loop(0, nc, chunk_body, 0)
            pltpu.async_copy(out_v.at[oslot],
                             out_hbm.at[i, pl.ds(base, _BPW)],
                             osem.at[oslot])
            pending_out[oslot] = i
        drain_out(0)
        drain_out(1)

    return k(*idx_list, *tables)


def _full_spec(shape):
    nd = len(shape)
    return pl.BlockSpec(shape, lambda i, _nd=nd: (0,) * _nd)


def _dot(a, b):
    return jnp.dot(a, b, preferred_element_type=jnp.float32,
                   precision=_PREC)


def _bottom_mlp(dense, bw0, bb0, bw1, bb1, bw2, bb2):
    bm = 512

    def body(x_ref, w0, b0, w1, b1, w2, b2, o_ref):
        h = jnp.maximum(_dot(x_ref[...], w0[...]) + b0[...], 0.0)
        h = jnp.maximum(_dot(h, w1[...]) + b1[...], 0.0)
        h = jnp.maximum(_dot(h, w2[...]) + b2[...], 0.0)
        o_ref[...] = h

    consts = (bw0, bb0.reshape(1, -1), bw1, bb1.reshape(1, -1),
              bw2, bb2.reshape(1, -1))
    return pl.pallas_call(
        body,
        grid=(_B // bm,),
        in_specs=[pl.BlockSpec((bm, 13), lambda i: (i, 0))]
        + [_full_spec(a.shape) for a in consts],
        out_specs=pl.BlockSpec((bm, _EMB), lambda i: (i, 0)),
        out_shape=jax.ShapeDtypeStruct((_B, _EMB), jnp.float32),
        compiler_params=_TC_PARAMS,
        interpret=_INTERP,
    )(dense, *consts)


def _dcn(demb, emb26, u0, v0, c0, u1, v1, c1, u2, v2, c2):
    bm = 256

    def body(demb_ref, emb_ref, u0r, v0r, c0r, u1r, v1r, c1r, u2r, v2r,
             c2r, o_ref):
        x0 = jnp.concatenate(
            [demb_ref[...]] + [emb_ref[t] for t in range(_NS)], axis=1)
        xl = x0
        for u_r, v_r, c_r in ((u0r, v0r, c0r), (u1r, v1r, c1r),
                              (u2r, v2r, c2r)):
            uu = _dot(xl, u_r[...])
            vv = _dot(uu, v_r[...]) + c_r[...]
            xl = x0 * vv + xl
        o_ref[...] = xl

    consts = (u0, v0, c0.reshape(1, -1), u1, v1, c1.reshape(1, -1),
              u2, v2, c2.reshape(1, -1))
    return pl.pallas_call(
        body,
        grid=(_B // bm,),
        in_specs=[
            pl.BlockSpec((bm, _EMB), lambda i: (i, 0)),
            pl.BlockSpec((_NS, bm, _EMB), lambda i: (0, i, 0)),
        ] + [_full_spec(a.shape) for a in consts],
        out_specs=pl.BlockSpec((bm, _D), lambda i: (i, 0)),
        out_shape=jax.ShapeDtypeStruct((_B, _D), jnp.float32),
        compiler_params=_TC_PARAMS,
        interpret=_INTERP,
    )(demb, emb26, *consts)


def _top_mlp(xl, tw0, tb0, tw1, tb1, tw2, tb2, tw3, tb3, tw4, tb4):
    bm = 512
    tw4p = jnp.pad(tw4, ((0, 0), (0, 127)))
    tb4p = jnp.pad(tb4.reshape(1, -1), ((0, 0), (0, 127)))

    def body(x_ref, w0, b0, w1, b1, w2, b2, w3, b3, w4, b4, o_ref):
        h = jnp.maximum(_dot(x_ref[...], w0[...]) + b0[...], 0.0)
        h = jnp.maximum(_dot(h, w1[...]) + b1[...], 0.0)
        h = jnp.maximum(_dot(h, w2[...]) + b2[...], 0.0)
        h = jnp.maximum(_dot(h, w3[...]) + b3[...], 0.0)
        o_ref[...] = _dot(h, w4[...]) + b4[...]

    consts = (tw0, tb0.reshape(1, -1), tw1, tb1.reshape(1, -1),
              tw2, tb2.reshape(1, -1), tw3, tb3.reshape(1, -1),
              tw4p, tb4p)
    return pl.pallas_call(
        body,
        grid=(_B // bm,),
        in_specs=[pl.BlockSpec((bm, _D), lambda i: (i, 0))]
        + [_full_spec(a.shape) for a in consts],
        out_specs=pl.BlockSpec((bm, _EMB), lambda i: (i, 0)),
        out_shape=jax.ShapeDtypeStruct((_B, _EMB), jnp.float32),
        compiler_params=_TC_PARAMS,
        interpret=_INTERP,
    )(xl, *consts)


def kernel(dense, sparse_0, sparse_1, sparse_2, sparse_3, sparse_4,
           sparse_5, sparse_6, sparse_7, sparse_8, sparse_9, sparse_10,
           sparse_11, sparse_12, sparse_13, sparse_14, sparse_15,
           sparse_16, sparse_17, sparse_18, sparse_19, sparse_20,
           sparse_21, sparse_22, sparse_23, sparse_24, sparse_25,
           table_0, table_1, table_2, table_3, table_4, table_5, table_6,
           table_7, table_8, table_9, table_10, table_11, table_12,
           table_13, table_14, table_15, table_16, table_17, table_18,
           table_19, table_20, table_21, table_22, table_23, table_24,
           table_25,
           bw_0, bb_0, bw_1, bb_1, bw_2, bb_2,
           u_0, v_0, dcnb_0, u_1, v_1, dcnb_1, u_2, v_2, dcnb_2,
           tw_0, tb_0, tw_1, tb_1, tw_2, tb_2, tw_3, tb_3, tw_4, tb_4):
    sparses = [sparse_0, sparse_1, sparse_2, sparse_3, sparse_4, sparse_5,
               sparse_6, sparse_7, sparse_8, sparse_9, sparse_10,
               sparse_11, sparse_12, sparse_13, sparse_14, sparse_15,
               sparse_16, sparse_17, sparse_18, sparse_19, sparse_20,
               sparse_21, sparse_22, sparse_23, sparse_24, sparse_25]
    tables = [table_0, table_1, table_2, table_3, table_4, table_5,
              table_6, table_7, table_8, table_9, table_10, table_11,
              table_12, table_13, table_14, table_15, table_16, table_17,
              table_18, table_19, table_20, table_21, table_22, table_23,
              table_24, table_25]
    idx = [s.reshape(_NW, nc, c * mh)
           for s, (c, nc), mh in zip(sparses, _CFG, _MH)]
    emb26 = _sc_embed(idx, tables)
    demb = _bottom_mlp(dense, bw_0, bb_0, bw_1, bb_1, bw_2, bb_2)
    xl = _dcn(demb, emb26, u_0, v_0, dcnb_0, u_1, v_1, dcnb_1,
              u_2, v_2, dcnb_2)
    top = _top_mlp(xl, tw_0, tb_0, tw_1, tb_1, tw_2, tb_2, tw_3, tb_3,
                   tw_4, tb_4)
    return top[:, 0]
